# baseline (speedup 1.0000x reference)
import functools

import jax
import jax.numpy as jnp
from jax import lax
from jax.experimental import pallas as pl
from jax.experimental.pallas import tpu as pltpu
from jax.experimental.pallas import tpu_sc as plsc

_NW = 32
_BLK = 320
_CH = 512
_NBUF = 4


def _wid():
    return lax.axis_index("s") * 2 + lax.axis_index("c")


def _convert_kernel(vocab, embed, w_in, w_out, vb0, vb1, vc0, vc1,
                    r0s, r1s, w0s, w1s):
    n_blocks = vocab // _BLK
    w = _wid()
    vbufs = (vb0, vb1)
    vcs = (vc0, vc1)
    rsems = (r0s, r1s)
    wsems = (w0s, w1s)

    def read(k, b):
        blk = w + k * _NW
        return pltpu.make_async_copy(
            w_in.at[pl.ds(blk * _BLK, _BLK), :], vbufs[b], rsems[b])

    def write(k, b):
        blk = w + k * _NW
        return pltpu.make_async_copy(
            vcs[b], w_out.at[pl.ds(blk * (_BLK // 2), _BLK // 2), :],
            wsems[b])

    def repack(b):
        def body(r2, carry):
            for half in range(2):
                for c in range(0, embed, 16):
                    vcs[b][r2, pl.ds(embed * half + c, 16)] = (
                        vbufs[b][2 * r2 + half, pl.ds(c, 16)])
            return carry
        lax.fori_loop(0, _BLK // 2, body, 0)

    n_k = pl.cdiv(n_blocks, _NW)

    @pl.when(w + 0 < n_blocks)
    def _():
        read(0, 0).start()

    @pl.when(w + _NW < n_blocks)
    def _():
        read(1, 1).start()

    for k in range(n_k):
        b = k % 2

        @pl.when(w + k * _NW < n_blocks)
        def _():
            read(k, b).wait()

            @pl.when((k >= 2) & (w + (k - 2) * _NW < n_blocks))
            def _():
                write(k - 2, b).wait()
            repack(b)
            write(k, b).start()

            @pl.when(w + (k + 2) * _NW < n_blocks)
            def _():
                read(k + 2, b).start()

    for k in (n_k - 3, n_k - 2, n_k - 1):
        @pl.when((w + k * _NW < n_blocks)
                 & ~(w + (k + 2) * _NW < n_blocks))
        def _(k=k):
            write(k, k % 2).wait()


def _gather_kernel(n_slots, embed, idx_hbm, table_hbm, out_hbm,
                   idx_v, rows, gsems, osems):
    per_w = n_slots // _NW
    n_chunks = per_w // _CH
    w = _wid()
    base = w * per_w
    pltpu.sync_copy(idx_hbm.at[pl.ds(base, per_w)], idx_v)

    def start_gather(c):
        b = c % 2
        return pltpu.async_copy(
            table_hbm.at[idx_v.at[pl.ds(c * _CH, _CH)]], rows[b], gsems[b])

    gcps = [start_gather(0), start_gather(1)]
    ocps = [None, None]
    for c in range(n_chunks):
        b = c % 2
        gcps[b].wait()
        ocps[b] = pltpu.async_copy(
            rows[b],
            out_hbm.at[pl.ds(base + c * _CH, _CH), pl.ds(0, embed)],
            osems[b])
        if c + 2 < n_chunks:
            ocps[b].wait()
            gcps[b] = start_gather(c + 2)
    ocps[0].wait()
    ocps[1].wait()


def kernel(x, weight):
    batch, fields = x.shape
    vocab, embed = weight.shape
    n_slots = batch * 32

    idx = jnp.pad(x, ((0, 0), (0, 32 - fields))).reshape(n_slots)

    mesh = plsc.VectorSubcoreMesh(core_axis_name="c", subcore_axis_name="s")

    convert = functools.partial(
        pl.kernel,
        mesh=mesh,
        out_type=jax.ShapeDtypeStruct((vocab // 2, 2 * embed), jnp.float32),
        scratch_types=[
            pltpu.VMEM((_BLK, embed), jnp.float32),
            pltpu.VMEM((_BLK, embed), jnp.float32),
            pltpu.VMEM((_BLK // 2, 2 * embed), jnp.float32),
            pltpu.VMEM((_BLK // 2, 2 * embed), jnp.float32),
            pltpu.SemaphoreType.DMA,
            pltpu.SemaphoreType.DMA,
            pltpu.SemaphoreType.DMA,
            pltpu.SemaphoreType.DMA,
        ],
        compiler_params=pltpu.CompilerParams(use_tc_tiling_on_sc=True),
    )(functools.partial(_convert_kernel, vocab, embed))
    wlin = convert(weight)
    wlin64 = wlin.reshape(vocab, embed)

    gather = functools.partial(
        pl.kernel,
        mesh=mesh,
        out_type=jax.ShapeDtypeStruct((n_slots, 2 * embed), jnp.float32),
        scratch_types=[
            pltpu.VMEM((n_slots // _NW,), jnp.int32),
            [pltpu.VMEM((_CH, embed), jnp.float32)] * 2,
            [pltpu.SemaphoreType.DMA] * 2,
            [pltpu.SemaphoreType.DMA] * 2,
        ],
        compiler_params=pltpu.CompilerParams(use_tc_tiling_on_sc=False),
    )(functools.partial(_gather_kernel, n_slots, embed))
    out128 = gather(idx, wlin64)

    return out128.reshape(batch, 32, 2 * embed)[:, :fields, :embed]


# speedup vs baseline: 2.6335x; 2.6335x over previous
import functools

import jax
import jax.numpy as jnp
from jax import lax
from jax.experimental import pallas as pl
from jax.experimental.pallas import tpu as pltpu
from jax.experimental.pallas import tpu_sc as plsc

_NW = 32
_BLK = 320
_CH = 512
_NBUF = 4


def _wid():
    return lax.axis_index("s") * 2 + lax.axis_index("c")


def _convert_kernel(vocab, embed, w_in, w_out, vb0, vb1, vc0, vc1,
                    r0s, r1s, w0s, w1s):
    n_blocks = vocab // _BLK
    w = _wid()
    vbufs = (vb0, vb1)
    vcs = (vc0, vc1)
    rsems = (r0s, r1s)
    wsems = (w0s, w1s)

    def read(k, b):
        blk = w + k * _NW
        return pltpu.make_async_copy(
            w_in.at[pl.ds(blk * _BLK, _BLK), :], vbufs[b], rsems[b])

    def write(k, b):
        blk = w + k * _NW
        return pltpu.make_async_copy(
            vcs[b], w_out.at[pl.ds(blk * (_BLK // 2), _BLK // 2), :],
            wsems[b])

    def repack(b):
        def body(r2, carry):
            for half in range(2):
                for c in range(0, embed, 16):
                    vcs[b][r2, pl.ds(embed * half + c, 16)] = (
                        vbufs[b][2 * r2 + half, pl.ds(c, 16)])
            return carry
        lax.fori_loop(0, _BLK // 2, body, 0)

    n_k = pl.cdiv(n_blocks, _NW)

    @pl.when(w + 0 < n_blocks)
    def _():
        read(0, 0).start()

    @pl.when(w + _NW < n_blocks)
    def _():
        read(1, 1).start()

    for k in range(n_k):
        b = k % 2

        @pl.when(w + k * _NW < n_blocks)
        def _():
            read(k, b).wait()

            @pl.when((k >= 2) & (w + (k - 2) * _NW < n_blocks))
            def _():
                write(k - 2, b).wait()
            repack(b)
            write(k, b).start()

            @pl.when(w + (k + 2) * _NW < n_blocks)
            def _():
                read(k + 2, b).start()

    for k in (n_k - 3, n_k - 2, n_k - 1):
        @pl.when((w + k * _NW < n_blocks)
                 & ~(w + (k + 2) * _NW < n_blocks))
        def _(k=k):
            write(k, k % 2).wait()


def _gather_kernel(n_slots, embed, idx_hbm, table_hbm, out_hbm,
                   idx_v, rows, gsems, osems):
    per_w = n_slots // _NW
    n_chunks = per_w // _CH
    w = _wid()
    base = w * per_w
    pltpu.sync_copy(idx_hbm.at[pl.ds(base, per_w)], idx_v)

    def start_gather(c):
        b = c % 2
        return pltpu.async_copy(
            table_hbm.at[idx_v.at[pl.ds(c * _CH, _CH)]], rows[b], gsems[b])

    gcps = [start_gather(0), start_gather(1)]
    ocps = [None, None]
    for c in range(n_chunks):
        b = c % 2
        gcps[b].wait()
        ocps[b] = pltpu.async_copy(
            rows[b],
            out_hbm.at[pl.ds(base + c * _CH, _CH), :],
            osems[b])
        if c + 2 < n_chunks:
            ocps[b].wait()
            gcps[b] = start_gather(c + 2)
    ocps[0].wait()
    ocps[1].wait()


def kernel(x, weight):
    batch, fields = x.shape
    vocab, embed = weight.shape
    n_slots = batch * fields

    idx = x.reshape(n_slots)

    mesh = plsc.VectorSubcoreMesh(core_axis_name="c", subcore_axis_name="s")

    convert = functools.partial(
        pl.kernel,
        mesh=mesh,
        out_type=jax.ShapeDtypeStruct((vocab // 2, 2 * embed), jnp.float32),
        scratch_types=[
            pltpu.VMEM((_BLK, embed), jnp.float32),
            pltpu.VMEM((_BLK, embed), jnp.float32),
            pltpu.VMEM((_BLK // 2, 2 * embed), jnp.float32),
            pltpu.VMEM((_BLK // 2, 2 * embed), jnp.float32),
            pltpu.SemaphoreType.DMA,
            pltpu.SemaphoreType.DMA,
            pltpu.SemaphoreType.DMA,
            pltpu.SemaphoreType.DMA,
        ],
        compiler_params=pltpu.CompilerParams(use_tc_tiling_on_sc=True),
    )(functools.partial(_convert_kernel, vocab, embed))
    wlin = convert(weight)
    wlin64 = wlin.reshape(vocab, embed)

    gather = functools.partial(
        pl.kernel,
        mesh=mesh,
        out_type=jax.ShapeDtypeStruct((n_slots, embed), jnp.float32),
        scratch_types=[
            pltpu.VMEM((n_slots // _NW,), jnp.int32),
            [pltpu.VMEM((_CH, embed), jnp.float32)] * 2,
            [pltpu.SemaphoreType.DMA] * 2,
            [pltpu.SemaphoreType.DMA] * 2,
        ],
        compiler_params=pltpu.CompilerParams(use_tc_tiling_on_sc=False),
    )(functools.partial(_gather_kernel, n_slots, embed))
    out = gather(idx, wlin64)

    return out.reshape(batch, fields, embed)


# speedup vs baseline: 3.1739x; 1.2052x over previous
import functools

import jax
import jax.numpy as jnp
from jax import lax
from jax.experimental import pallas as pl
from jax.experimental.pallas import tpu as pltpu
from jax.experimental.pallas import tpu_sc as plsc


def _gather_kernel(n_total, n_chunks, chunk, idx_hbm, table_hbm,
                   out_hbm, idx_v, rows0, rows1, g0, g1, o0, o1):
    num_cores = 2
    wid = lax.axis_index("s") * num_cores + lax.axis_index("c")
    per_w = n_total // 32
    base = wid * per_w
    pltpu.sync_copy(idx_hbm.at[pl.ds(base, per_w)], idx_v)

    bufs = (rows0, rows1)
    gsems = (g0, g1)
    osems = (o0, o1)

    def start_gather(i):
        b = i % 2
        return pltpu.async_copy(
            table_hbm.at[idx_v.at[pl.ds(i * chunk, chunk)]],
            bufs[b], gsems[b])

    gcps = [start_gather(0), start_gather(1)]
    ocps = [None, None]
    for i in range(n_chunks):
        b = i % 2
        gcps[b].wait()
        ocps[b] = pltpu.async_copy(
            bufs[b], out_hbm.at[pl.ds(base + i * chunk, chunk)], osems[b])
        if i + 2 < n_chunks:
            ocps[b].wait()
            gcps[b] = start_gather(i + 2)
    ocps[0].wait()
    ocps[1].wait()


def kernel(x, weight):
    batch, fields = x.shape
    vocab, embed = weight.shape
    n_total = batch * fields
    n_workers = 32
    per_w = n_total // n_workers
    chunk = 832
    n_chunks = per_w // chunk

    idx = x.reshape(n_total)

    mesh = plsc.VectorSubcoreMesh(core_axis_name="c", subcore_axis_name="s")
    run = functools.partial(
        pl.kernel,
        mesh=mesh,
        out_type=jax.ShapeDtypeStruct((n_total, embed), jnp.float32),
        scratch_types=[
            pltpu.VMEM((per_w,), jnp.int32),
            pltpu.VMEM((chunk, embed), jnp.float32),
            pltpu.VMEM((chunk, embed), jnp.float32),
            pltpu.SemaphoreType.DMA,
            pltpu.SemaphoreType.DMA,
            pltpu.SemaphoreType.DMA,
            pltpu.SemaphoreType.DMA,
        ],
        compiler_params=pltpu.CompilerParams(use_tc_tiling_on_sc=False),
    )(functools.partial(_gather_kernel, n_total, n_chunks, chunk))

    out = run(idx, weight)
    return out.reshape(batch, fields, embed)
